# baseline (speedup 1.0000x reference)
import jax
import jax.numpy as jnp
from jax import lax
from jax.experimental import pallas as pl
from jax.experimental.pallas import tpu as pltpu
from jax.experimental.pallas import tpu_sc as plsc

VOCAB = 1000000
DIM = 64
B = 16384

NUM_CORES = 2
NUM_SUBCORES = 16
LANES = 16
NW = NUM_CORES * NUM_SUBCORES
BPW = B // NW

CHK = 1024
VTAIL = 999424
NCHK = VTAIL // CHK
CMAX = 31
NAUX = VOCAB - VTAIL
NG = 4
GCAP = NG * LANES
CAP = 768
NEDGE = 992
DUMP = B
IROWS = B + 16


def _sweep_body(st_hbm, pt_hbm, sc_hbm, pc_hbm, bt_hbm, bc_hbm, table_hbm,
                u2_hbm, v2_hbm,
                btv, bcv, sval, pval, cval, qval,
                win, rst0, rst1, rsc0, rsc1, pt0, pt1, pc0, pc1,
                semw, sems):
    wid = lax.axis_index("s") * NUM_CORES + lax.axis_index("c")
    c0 = (NCHK * wid) // NW
    nchk = (NCHK * (wid + 1)) // NW - c0

    pltpu.sync_copy(bt_hbm, btv)
    pltpu.sync_copy(bc_hbm, bcv)

    def stage(bv, v_hbm, p_hbm, vdst, pdst):
        s16 = bv[pl.ds(c0, LANES)]
        off = jnp.minimum((s16[0] // 8) * 8, B - CAP)
        off = pl.multiple_of(off, 8)
        pltpu.sync_copy(v_hbm.at[pl.ds(off, CAP)], vdst)
        pltpu.sync_copy(p_hbm.at[pl.ds(off, CAP)], pdst)
        return off

    toff = stage(btv, st_hbm, pt_hbm, sval, pval)
    coff = stage(bcv, sc_hbm, pc_hbm, cval, qval)

    lanes = lax.iota(jnp.int32, LANES)
    rstages = [(rst0, rsc0, pt0, pc0), (rst1, rsc1, pt1, pc1)]
    dumpvec = jnp.full((LANES,), DUMP, jnp.int32)

    def fire_slabs(k):
        c = c0 + jnp.minimum(k, nchk - 1)
        base = pl.multiple_of(c * CHK, 128)
        hs = []
        for tr in range(DIM // 8):
            hs.append(pltpu.async_copy(
                table_hbm.at[pl.ds(tr * 8, 8), pl.ds(base, CHK)],
                win.at[pl.ds(tr * 8, 8), pl.ds(0, CHK)], semw))
        return hs

    def chunk_compute(k, rt, rc, pbt, pbc):
        c = c0 + jnp.minimum(k, nchk - 1)
        base = c * CHK
        for pb in (pbt, pbc):
            for q in range(NG):
                pb[pl.ds(q * LANES, LANES)] = dumpvec

        def one_table(bv, vals, poss, off, rstage, pb):
            e16 = bv[pl.ds(c, LANES)]
            s = e16[0]
            e = e16[1]

            def group(g, carry):
                i = s + g * LANES

                @pl.when(i < e)
                def _():
                    li = i - off
                    sv16 = vals[pl.ds(li, LANES)]
                    sp16 = poss[pl.ds(li, LANES)]
                    msk = (lanes + i) < e
                    cols = jnp.clip(sv16 - base, 0, CHK - 1)
                    pb[pl.ds(g * LANES, LANES)] = jnp.where(msk, sp16, DUMP)
                    rows16 = g * LANES + lanes

                    def dim_step(d, dc):
                        dsp = jnp.full((LANES,), 1, jnp.int32) * d
                        gv = plsc.load_gather(win, [dsp, cols])
                        plsc.store_scatter(rstage, [rows16, dsp], gv)
                        return dc

                    lax.fori_loop(0, DIM, dim_step, 0)
                return carry

            lax.fori_loop(0, NG, group, 0)

        one_table(btv, sval, pval, toff, rt, pbt)
        one_table(bcv, cval, qval, coff, rc, pbc)
        hu = pltpu.async_copy(rt, u2_hbm.at[pbt], sems)
        hv = pltpu.async_copy(rc, v2_hbm.at[pbc], sems)
        return hu, hv

    slab_hs = fire_slabs(0)
    scat_hs = [None, None]
    for k in range(CMAX):
        for h in slab_hs:
            h.wait()
        par = k % 2
        if scat_hs[par] is not None:
            for h in scat_hs[par]:
                h.wait()
        rt, rc, pbt, pbc = rstages[par]
        hu, hv = chunk_compute(k, rt, rc, pbt, pbc)
        scat_hs[par] = (hu, hv)
        if k + 1 < CMAX:
            slab_hs = fire_slabs(k + 1)
    for hs in scat_hs:
        if hs is not None:
            for h in hs:
                h.wait()


def _dot_body(t_hbm, c_hbm, u2_hbm, v2_hbm, aux_hbm, out_hbm,
              tv, cv, ub, vb, auxv, out_v, sem):
    wid = lax.axis_index("s") * NUM_CORES + lax.axis_index("c")
    base = wid * BPW

    pltpu.sync_copy(t_hbm.at[pl.ds(base, BPW)], tv)
    pltpu.sync_copy(c_hbm.at[pl.ds(base, BPW)], cv)
    pltpu.sync_copy(aux_hbm, auxv)

    lanes = lax.iota(jnp.int32, LANES)
    NCH = BPW // 128

    for ch in range(NCH):
        row0 = pl.multiple_of(base + ch * 128, 8)
        pltpu.sync_copy(u2_hbm.at[pl.ds(row0, 128), pl.ds(0, 2 * DIM)], ub)
        pltpu.sync_copy(v2_hbm.at[pl.ds(row0, 128), pl.ds(0, 2 * DIM)], vb)

        def group(g, carry):
            i0 = ch * 128 + g * LANES
            t16 = tv[pl.ds(i0, LANES)]
            c16 = cv[pl.ds(i0, LANES)]
            tm = t16 >= VTAIL
            cm = c16 >= VTAIL
            ta = jnp.clip(t16 - VTAIL, 0, NAUX - 1)
            ca = jnp.clip(c16 - VTAIL, 0, NAUX - 1)
            rloc = lanes + g * LANES
            acc = jnp.zeros((LANES,), jnp.float32)
            for d in range(DIM):
                dsp = jnp.full((LANES,), d, jnp.int32)
                uu = plsc.load_gather(ub, [rloc, dsp])
                vv = plsc.load_gather(vb, [rloc, dsp])
                au = plsc.load_gather(auxv, [ta, dsp])
                av = plsc.load_gather(auxv, [ca, dsp])
                u = jnp.where(tm, au, uu)
                v = jnp.where(cm, av, vv)
                acc = acc + u * v
            out_v[pl.ds(i0, LANES)] = acc
            return carry

        lax.fori_loop(0, 8, group, 0)

    pltpu.sync_copy(out_v, out_hbm.at[pl.ds(base, BPW)])


@jax.jit
def _skipgram(t32, c32, table_t, aux):
    pt = jnp.argsort(t32).astype(jnp.int32)
    st = t32[pt]
    pc = jnp.argsort(c32).astype(jnp.int32)
    sc_ = c32[pc]
    edges = jnp.minimum(jnp.arange(NEDGE, dtype=jnp.int32) * CHK, VOCAB)
    bt = jnp.searchsorted(st, edges, method="sort").astype(jnp.int32)
    bc = jnp.searchsorted(sc_, edges, method="sort").astype(jnp.int32)

    mesh = plsc.VectorSubcoreMesh(core_axis_name="c", subcore_axis_name="s")
    u2, v2 = pl.kernel(
        _sweep_body,
        out_type=(jax.ShapeDtypeStruct((IROWS, 2 * DIM), jnp.float32),
                  jax.ShapeDtypeStruct((IROWS, 2 * DIM), jnp.float32)),
        mesh=mesh,
        scratch_types=[
            pltpu.VMEM((NEDGE,), jnp.int32),
            pltpu.VMEM((NEDGE,), jnp.int32),
            pltpu.VMEM((CAP,), jnp.int32),
            pltpu.VMEM((CAP,), jnp.int32),
            pltpu.VMEM((CAP,), jnp.int32),
            pltpu.VMEM((CAP,), jnp.int32),
            pltpu.VMEM((DIM, CHK), jnp.float32),
            pltpu.VMEM((GCAP, 2 * DIM), jnp.float32),
            pltpu.VMEM((GCAP, 2 * DIM), jnp.float32),
            pltpu.VMEM((GCAP, 2 * DIM), jnp.float32),
            pltpu.VMEM((GCAP, 2 * DIM), jnp.float32),
            pltpu.VMEM((GCAP,), jnp.int32),
            pltpu.VMEM((GCAP,), jnp.int32),
            pltpu.VMEM((GCAP,), jnp.int32),
            pltpu.VMEM((GCAP,), jnp.int32),
            pltpu.SemaphoreType.DMA,
            pltpu.SemaphoreType.DMA,
        ],
        compiler_params=pltpu.CompilerParams(needs_layout_passes=False),
    )(st, pt, sc_, pc, bt, bc, table_t)

    return pl.kernel(
        _dot_body,
        out_type=jax.ShapeDtypeStruct((B,), jnp.float32),
        mesh=mesh,
        scratch_types=[
            pltpu.VMEM((BPW,), jnp.int32),
            pltpu.VMEM((BPW,), jnp.int32),
            pltpu.VMEM((128, 2 * DIM), jnp.float32),
            pltpu.VMEM((128, 2 * DIM), jnp.float32),
            pltpu.VMEM((NAUX, 2 * DIM), jnp.float32),
            pltpu.VMEM((BPW,), jnp.float32),
            pltpu.SemaphoreType.DMA,
        ],
        compiler_params=pltpu.CompilerParams(needs_layout_passes=False),
    )(t32, c32, u2, v2, aux)


def kernel(target, context, embedding_weights):
    t32 = target.astype(jnp.int32)
    c32 = context.astype(jnp.int32)
    aux = jnp.pad(embedding_weights[VTAIL:], ((0, 0), (0, DIM)))
    return _skipgram(t32, c32, embedding_weights.T, aux)
